# baseline (speedup 1.0000x reference)
import jax
import jax.numpy as jnp
from jax import lax
from jax.experimental import pallas as pl
from jax.experimental.pallas import tpu as pltpu
from jax.experimental.pallas import tpu_sc as plsc

N_PIX = 8192
EMB = 1024
B = 4
B_SC = 1
B_TC = B - B_SC

NC = 2
NS = 16
NW = NC * NS

R = 16
ROWS_PER_W = N_PIX // NW
NCH = ROWS_PER_W // R

BLK = 512


def _sc_body(x_hbm, pos_hbm, out_hbm, p0, p1, xb0, xb1, si0, si1, so0, so1):
    wid = lax.axis_index("s") * NC + lax.axis_index("c")
    row_base = wid * ROWS_PER_W
    pbufs = (p0, p1)
    xbufs = (xb0, xb1)
    isems = (si0, si1)
    osems = (so0, so1)

    def in_copies(c, s):
        r0 = row_base + c * R
        return (
            pltpu.make_async_copy(pos_hbm.at[pl.ds(r0, R), :], pbufs[s], isems[s]),
            pltpu.make_async_copy(x_hbm.at[B_TC, pl.ds(r0, R), :], xbufs[s], isems[s]),
        )

    def out_copy(c, s):
        r0 = row_base + c * R
        return pltpu.make_async_copy(
            xbufs[s], out_hbm.at[pl.ds(r0, R), :], osems[s])

    def start_in(c, s):
        for cp in in_copies(c, s):
            cp.start()

    def wait_in(c, s):
        for cp in in_copies(c, s):
            cp.wait()

    def compute(s):
        pv_ref, xv_ref = pbufs[s], xbufs[s]

        def vec(i, _):
            r = i // (EMB // 16)
            off = (i % (EMB // 16)) * 16
            sl = pl.ds(off, 16)
            xv_ref[r, sl] = xv_ref[r, sl] + pv_ref[r, sl]
            return ()

        lax.fori_loop(0, R * EMB // 16, vec, (), unroll=8)

    start_in(0, 0)
    start_in(1, 1)
    wait_in(0, 0)
    compute(0)
    out_copy(0, 0).start()

    def pair(k, _):
        for s in (1, 0):
            c = 2 * k + 1 + (1 - s)
            out_copy(c - 1, 1 - s).wait()
            start_in(c + 1, 1 - s)
            wait_in(c, s)
            compute(s)
            out_copy(c, s).start()
        return ()

    lax.fori_loop(0, (NCH - 2) // 2, pair, ())

    c = NCH - 1
    out_copy(c - 1, 0).wait()
    wait_in(c, 1)
    compute(1)
    out_copy(c, 1).start()
    out_copy(c, 1).wait()


def _sc_add(x, pos_table):
    mesh = plsc.VectorSubcoreMesh(core_axis_name="c", subcore_axis_name="s")
    f = pl.kernel(
        _sc_body,
        out_type=jax.ShapeDtypeStruct((N_PIX, EMB), jnp.float32),
        mesh=mesh,
        scratch_types=[
            pltpu.VMEM((R, EMB), jnp.float32),
            pltpu.VMEM((R, EMB), jnp.float32),
            pltpu.VMEM((R, EMB), jnp.float32),
            pltpu.VMEM((R, EMB), jnp.float32),
            pltpu.SemaphoreType.DMA,
            pltpu.SemaphoreType.DMA,
            pltpu.SemaphoreType.DMA,
            pltpu.SemaphoreType.DMA,
        ],
    )
    return f(x, pos_table)


def _tc_kernel(x_ref, pos_ref, o_ref):
    o_ref[...] = x_ref[...] + pos_ref[...][None, :, :]


def _tc_add(x, pos_table):
    return pl.pallas_call(
        _tc_kernel,
        grid=(N_PIX // BLK,),
        in_specs=[
            pl.BlockSpec((B_TC, BLK, EMB), lambda i: (0, i, 0)),
            pl.BlockSpec((BLK, EMB), lambda i: (i, 0)),
        ],
        out_specs=pl.BlockSpec((B_TC, BLK, EMB), lambda i: (0, i, 0)),
        out_shape=jax.ShapeDtypeStruct((B_TC, N_PIX, EMB), jnp.float32),
    )(x, pos_table)


def kernel(x, pos_table):
    out_sc = _sc_add(x, pos_table)
    out_tc = _tc_add(x, pos_table)
    return jnp.concatenate([out_tc, out_sc[None]], axis=0)


# speedup vs baseline: 2.1291x; 2.1291x over previous
import jax
import jax.numpy as jnp
from jax.experimental import pallas as pl

N_PIX = 8192
EMB = 1024
B = 4
BLK = 1024


def _add_kernel(x_ref, pos_ref, o_ref):
    o_ref[...] = x_ref[...] + pos_ref[...]


def kernel(x, pos_table):
    x2 = x.reshape(B * N_PIX, EMB)
    out = pl.pallas_call(
        _add_kernel,
        grid=(N_PIX // BLK, B),
        in_specs=[
            pl.BlockSpec((BLK, EMB), lambda i, j: (j * (N_PIX // BLK) + i, 0)),
            pl.BlockSpec((BLK, EMB), lambda i, j: (i, 0)),
        ],
        out_specs=pl.BlockSpec((BLK, EMB), lambda i, j: (j * (N_PIX // BLK) + i, 0)),
        out_shape=jax.ShapeDtypeStruct((B * N_PIX, EMB), jnp.float32),
    )(x2, pos_table)
    return out.reshape(B, N_PIX, EMB)


# speedup vs baseline: 2.2093x; 1.0376x over previous
import jax
import jax.numpy as jnp
from jax.experimental import pallas as pl

N_PIX = 8192
EMB = 1024
B = 4
BLK = 2048


def _add_kernel(x_ref, pos_ref, o_ref):
    o_ref[...] = x_ref[...] + pos_ref[...]


def kernel(x, pos_table):
    x2 = x.reshape(B * N_PIX, EMB)
    out = pl.pallas_call(
        _add_kernel,
        grid=(N_PIX // BLK, B),
        in_specs=[
            pl.BlockSpec((BLK, EMB), lambda i, j: (j * (N_PIX // BLK) + i, 0)),
            pl.BlockSpec((BLK, EMB), lambda i, j: (i, 0)),
        ],
        out_specs=pl.BlockSpec((BLK, EMB), lambda i, j: (j * (N_PIX // BLK) + i, 0)),
        out_shape=jax.ShapeDtypeStruct((B * N_PIX, EMB), jnp.float32),
    )(x2, pos_table)
    return out.reshape(B, N_PIX, EMB)
